# baseline (speedup 1.0000x reference)
import functools

import jax
import jax.numpy as jnp
from jax import lax
from jax.experimental import pallas as pl
from jax.experimental.pallas import tpu as pltpu
from jax.experimental.pallas import tpu_sc as plsc

DIM = 128
NC, NS = 2, 16
NW = NC * NS
CHUNK = 128


def _fold_kernel(emb_ref, w_ref, b_ref, out_ref):
    out_ref[...] = lax.dot_general(
        emb_ref[...], w_ref[...],
        dimension_numbers=(((1,), (1,)), ((), ())),
        preferred_element_type=jnp.float32,
    ) + b_ref[...]


def _fold_table(emb_pad, W, b):
    rows = emb_pad.shape[0]
    return pl.pallas_call(
        _fold_kernel,
        out_shape=jax.ShapeDtypeStruct((rows, DIM), jnp.float32),
    )(emb_pad, W, b.reshape(1, DIM))


@functools.lru_cache(maxsize=None)
def _make_gather(n_total, rows):
    assert n_total % (NW * CHUNK) == 0
    per_w = n_total // NW
    n_chunks = per_w // CHUNK
    mesh = plsc.VectorSubcoreMesh(
        core_axis_name="c", subcore_axis_name="s",
        num_cores=NC, num_subcores=NS)

    @functools.partial(
        pl.kernel, mesh=mesh,
        out_type=jax.ShapeDtypeStruct((n_total, DIM), jnp.float32),
        scratch_types=[
            pltpu.VMEM((CHUNK,), jnp.int32),
            pltpu.VMEM((CHUNK, DIM), jnp.float32),
            pltpu.SemaphoreType.DMA,
        ],
    )
    def gather(table_hbm, idx_hbm, out_hbm, idx_v, rows_v, sem):
        wid = lax.axis_index("s") * NC + lax.axis_index("c")
        base = wid * per_w

        @pl.loop(0, n_chunks)
        def _(t):
            off = base + t * CHUNK
            pltpu.sync_copy(idx_hbm.at[pl.ds(off, CHUNK)], idx_v)
            pltpu.async_copy(table_hbm.at[idx_v], rows_v, sem).wait()
            pltpu.sync_copy(rows_v, out_hbm.at[pl.ds(off, CHUNK)])

    return gather


def kernel(x, embed_table, W, b):
    B, C, H, W_ = x.shape
    L = C * H * W_
    idx = x.reshape(-1).astype(jnp.int32)
    vocab = embed_table.shape[0]
    rows = max(8, -(-vocab // 8) * 8)
    emb_pad = jnp.zeros((rows, DIM), embed_table.dtype).at[:vocab].set(embed_table)
    proj = _fold_table(emb_pad, W, b)
    out = _make_gather(idx.shape[0], rows)(proj, idx)
    return out.reshape(B, L, DIM)


# speedup vs baseline: 1.0030x; 1.0030x over previous
import functools

import jax
import jax.numpy as jnp
from jax import lax
from jax.experimental import pallas as pl
from jax.experimental.pallas import tpu as pltpu
from jax.experimental.pallas import tpu_sc as plsc

DIM = 128
NC, NS = 2, 16
NW = NC * NS
CHUNK = 64


def _fold_kernel(emb_ref, w_ref, b_ref, out_ref):
    out_ref[...] = lax.dot_general(
        emb_ref[...], w_ref[...],
        dimension_numbers=(((1,), (1,)), ((), ())),
        preferred_element_type=jnp.float32,
    ) + b_ref[...]


def _fold_table(emb_pad, W, b):
    rows = emb_pad.shape[0]
    return pl.pallas_call(
        _fold_kernel,
        out_shape=jax.ShapeDtypeStruct((rows, DIM), jnp.float32),
    )(emb_pad, W, b.reshape(1, DIM))


GROUP = 4


@functools.lru_cache(maxsize=None)
def _make_gather(n_total, rows):
    assert n_total % (NW * CHUNK) == 0
    per_w = n_total // NW
    n_chunks = per_w // CHUNK
    n_groups = n_chunks // GROUP
    assert n_chunks % GROUP == 0 and n_groups % 2 == 0 and n_groups >= 4
    mesh = plsc.VectorSubcoreMesh(
        core_axis_name="c", subcore_axis_name="s",
        num_cores=NC, num_subcores=NS)

    @functools.partial(
        pl.kernel, mesh=mesh,
        out_type=jax.ShapeDtypeStruct((n_total, DIM), jnp.float32),
        scratch_types=[
            pltpu.VMEM((per_w,), jnp.int32),
            pltpu.VMEM((GROUP, CHUNK, DIM), jnp.float32),
            pltpu.VMEM((GROUP, CHUNK, DIM), jnp.float32),
            pltpu.SemaphoreType.DMA,
            pltpu.SemaphoreType.DMA,
            pltpu.SemaphoreType.DMA,
        ],
    )
    def gather(table_hbm, idx_hbm, out_hbm, idx_v, buf_a, buf_b, gsem, sa, sb):
        wid = lax.axis_index("s") * NC + lax.axis_index("c")
        base = wid * per_w
        pltpu.sync_copy(idx_hbm.at[pl.ds(base, per_w)], idx_v)

        def fill(buf, g):
            descs = []
            for j in range(GROUP):
                ii = (g * GROUP + j) * CHUNK
                descs.append(pltpu.async_copy(
                    table_hbm.at[idx_v.at[pl.ds(ii, CHUNK)]], buf.at[j], gsem))
            for d in descs:
                d.wait()

        def store_start(buf, g, sem):
            descs = []
            for j in range(GROUP):
                off = base + (g * GROUP + j) * CHUNK
                descs.append(pltpu.async_copy(
                    buf.at[j], out_hbm.at[pl.ds(off, CHUNK)], sem))
            return descs

        def drain(descs):
            for d in descs:
                d.wait()

        fill(buf_a, 0)

        @pl.loop(0, n_groups // 2 - 1)
        def _(i):
            g0 = 2 * i
            da = store_start(buf_a, g0, sa)
            fill(buf_b, g0 + 1)
            drain(da)
            db = store_start(buf_b, g0 + 1, sb)
            fill(buf_a, g0 + 2)
            drain(db)

        da = store_start(buf_a, n_groups - 2, sa)
        fill(buf_b, n_groups - 1)
        drain(da)
        db = store_start(buf_b, n_groups - 1, sb)
        drain(db)

    return gather


def kernel(x, embed_table, W, b):
    B, C, H, W_ = x.shape
    L = C * H * W_
    idx = x.reshape(-1).astype(jnp.int32)
    vocab = embed_table.shape[0]
    rows = max(8, -(-vocab // 8) * 8)
    emb_pad = jnp.zeros((rows, DIM), embed_table.dtype).at[:vocab].set(embed_table)
    proj = _fold_table(emb_pad, W, b)
    out = _make_gather(idx.shape[0], rows)(proj, idx)
    return out.reshape(B, L, DIM)
